# baseline (speedup 1.0000x reference)
import jax
import jax.numpy as jnp
from jax.experimental import pallas as pl

B, T, H = 64, 2048, 256
NBINS, OUT, FILT, K = 256, 256, 256, 3
_EPS = 1e-5


def _im2col3(x):
    z = jnp.zeros((1, x.shape[1]), x.dtype)
    prv = jnp.concatenate([z, x[:-1]], axis=0)
    nxt = jnp.concatenate([x[1:], z], axis=0)
    return jnp.concatenate([prv, x, nxt], axis=1)


def _ln(h, g_row, b_row):
    mu = jnp.mean(h, axis=-1, keepdims=True)
    d = h - mu
    var = jnp.mean(d * d, axis=-1, keepdims=True)
    return g_row * d * jax.lax.rsqrt(var + _EPS) + b_row


def _onehot_lookup(v_col, lo_row, hi_row, emb_ref):
    oh = ((v_col > lo_row) & (v_col <= hi_row)).astype(jnp.float32)
    return jnp.dot(oh, emb_ref[:, :], preferred_element_type=jnp.float32)


def _va_kernel(x_ref, mask_ref, pt_ref, et_ref,
               plo_ref, phi_ref, elo_ref, ehi_ref,
               pemb_ref, eemb_ref,
               w1_ref, b1_ref, g1_ref, bt1_ref,
               p_w2, p_b2, p_g2, p_bt2, p_lw, p_lb,
               e_w2, e_b2, e_g2, e_bt2, e_lw, e_lb,
               ppred_ref, pembo_ref, epred_ref, eembo_ref):
    bf16 = jnp.bfloat16
    x = x_ref[0].astype(bf16)
    mask = mask_ref[0]

    xs = _im2col3(x)
    h12 = jnp.dot(xs, w1_ref[:, :], preferred_element_type=jnp.float32)
    h12 = jax.nn.relu(h12 + b1_ref[:, :])
    h_p = _ln(h12[:, :FILT], g1_ref[:, :FILT], bt1_ref[:, :FILT])
    h_e = _ln(h12[:, FILT:], g1_ref[:, FILT:], bt1_ref[:, FILT:])

    def head(h, w2, b2, g2, bt2, lw, lb):
        hs = _im2col3(h.astype(bf16))
        h2 = jnp.dot(hs, w2[:, :], preferred_element_type=jnp.float32)
        h2 = _ln(jax.nn.relu(h2 + b2[:, :]), g2[:, :], bt2[:, :])
        pred = jnp.sum(h2 * lw[:, :], axis=-1, keepdims=True) + lb[0, 0]
        return jnp.where(mask > 0.0, 0.0, pred)

    ppred_ref[0] = head(h_p, p_w2, p_b2, p_g2, p_bt2, p_lw, p_lb)
    epred_ref[0] = head(h_e, e_w2, e_b2, e_g2, e_bt2, e_lw, e_lb)

    pembo_ref[0] = _onehot_lookup(pt_ref[0], plo_ref[:, :], phi_ref[:, :],
                                  pemb_ref)
    eembo_ref[0] = _onehot_lookup(et_ref[0], elo_ref[:, :], ehi_ref[:, :],
                                  eemb_ref)


def _row2(a):
    return a.reshape(1, -1)


def kernel(embeddings, src_mask, pitch_target, energy_target, pitch_bins,
           energy_bins, pitch_emb, energy_emb, p_params, e_params):
    f32, bf16 = jnp.float32, jnp.bfloat16
    mask_f = src_mask.astype(f32).reshape(B, T, 1)
    pt = pitch_target.reshape(B, T, 1)
    et = energy_target.reshape(B, T, 1)

    inf = jnp.full((1,), jnp.inf, f32)
    plo = jnp.concatenate([-inf, pitch_bins]).reshape(1, NBINS)
    phi = jnp.concatenate([pitch_bins, inf]).reshape(1, NBINS)
    elo = jnp.concatenate([-inf, energy_bins]).reshape(1, NBINS)
    ehi = jnp.concatenate([energy_bins, inf]).reshape(1, NBINS)

    w1 = jnp.concatenate([p_params["conv1_w"].reshape(K * H, FILT),
                          e_params["conv1_w"].reshape(K * H, FILT)],
                         axis=1).astype(bf16)
    b1 = jnp.concatenate([p_params["conv1_b"], e_params["conv1_b"]])
    g1 = jnp.concatenate([p_params["ln1_g"], e_params["ln1_g"]])
    bt1 = jnp.concatenate([p_params["ln1_b"], e_params["ln1_b"]])

    def head_params(p):
        return (p["conv2_w"].reshape(K * FILT, FILT).astype(bf16),
                _row2(p["conv2_b"]), _row2(p["ln2_g"]), _row2(p["ln2_b"]),
                p["lin_w"].reshape(1, FILT), p["lin_b"].reshape(1, 1))

    whole = lambda shape: pl.BlockSpec(shape, lambda i: (0,) * len(shape))
    per_b3 = lambda shape: pl.BlockSpec(shape, lambda i: (i, 0, 0))

    in_specs = (
        [per_b3((1, T, H)), per_b3((1, T, 1)), per_b3((1, T, 1)),
         per_b3((1, T, 1))]
        + [whole((1, NBINS))] * 4
        + [whole((NBINS, OUT))] * 2
        + [whole((K * H, 2 * FILT)), whole((1, 2 * FILT)),
           whole((1, 2 * FILT)), whole((1, 2 * FILT))]
        + [whole((K * FILT, FILT)), whole((1, FILT)), whole((1, FILT)),
           whole((1, FILT)), whole((1, FILT)), whole((1, 1))] * 2
    )
    out_specs = [per_b3((1, T, 1)), per_b3((1, T, OUT)),
                 per_b3((1, T, 1)), per_b3((1, T, OUT))]
    out_shape = [jax.ShapeDtypeStruct((B, T, 1), f32),
                 jax.ShapeDtypeStruct((B, T, OUT), f32),
                 jax.ShapeDtypeStruct((B, T, 1), f32),
                 jax.ShapeDtypeStruct((B, T, OUT), f32)]

    ppred, pembo, epred, eembo = pl.pallas_call(
        _va_kernel,
        grid=(B,),
        in_specs=in_specs,
        out_specs=out_specs,
        out_shape=out_shape,
    )(embeddings, mask_f, pt, et, plo, phi, elo, ehi, pitch_emb, energy_emb,
      w1, _row2(b1), _row2(g1), _row2(bt1),
      *head_params(p_params), *head_params(e_params))

    return (ppred.reshape(B, T), pembo, epred.reshape(B, T), eembo)


# speedup vs baseline: 1.0437x; 1.0437x over previous
import jax
import jax.numpy as jnp
from jax.experimental import pallas as pl

B, T, H = 64, 2048, 256
NBINS, OUT, FILT, K = 256, 256, 256, 3
_EPS = 1e-5


def _im2col3(x, pad_row):
    prv = jnp.concatenate([pad_row, x[:-1]], axis=0)
    nxt = jnp.concatenate([x[1:], pad_row], axis=0)
    return jnp.concatenate([prv, x, nxt], axis=1)


def _rowstats(h):
    mu = jnp.mean(h, axis=-1, keepdims=True)
    m2 = jnp.mean(h * h, axis=-1, keepdims=True)
    return mu, jax.lax.rsqrt(m2 - mu * mu + _EPS)


def _onehot_lookup(v_col, lo_row, hi_row, emb_ref):
    oh = ((v_col > lo_row) & (v_col <= hi_row)).astype(jnp.bfloat16)
    return jnp.dot(oh, emb_ref[:, :], preferred_element_type=jnp.float32)


def _va_kernel(x_ref, mask_ref, pt_ref, et_ref,
               plo_ref, phi_ref, elo_ref, ehi_ref,
               pemb_ref, eemb_ref,
               w1_ref, b1_ref,
               p_w2, p_b2, p_pad, p_lwg, p_sc,
               e_w2, e_b2, e_pad, e_lwg, e_sc,
               ppred_ref, pembo_ref, epred_ref, eembo_ref):
    bf16 = jnp.bfloat16
    x = x_ref[0].astype(bf16)
    mask = mask_ref[0]
    zrow = jnp.zeros((1, H), bf16)

    xs = _im2col3(x, zrow)
    h12 = jnp.dot(xs, w1_ref[:, :], preferred_element_type=jnp.float32)
    h12 = jax.nn.relu(h12 + b1_ref[:, :])

    def head(h, w2, b2, pad, lwg, sc):
        mu, r = _rowstats(h)
        z = (h * r - mu * r).astype(bf16)
        zim = _im2col3(z, pad[:, :].astype(bf16))
        h2 = jnp.dot(zim, w2[:, :], preferred_element_type=jnp.float32)
        h2 = jax.nn.relu(h2 + b2[:, :])
        mu2, r2 = _rowstats(h2)
        s1 = jnp.sum(h2 * lwg[:, :], axis=-1, keepdims=True)
        pred = r2 * (s1 - mu2 * sc[0, 0]) + sc[0, 1]
        return jnp.where(mask > 0.0, 0.0, pred)

    ppred_ref[0] = head(h12[:, :FILT], p_w2, p_b2, p_pad, p_lwg, p_sc)
    epred_ref[0] = head(h12[:, FILT:], e_w2, e_b2, e_pad, e_lwg, e_sc)

    pembo_ref[0] = _onehot_lookup(pt_ref[0], plo_ref[:, :], phi_ref[:, :],
                                  pemb_ref)
    eembo_ref[0] = _onehot_lookup(et_ref[0], elo_ref[:, :], ehi_ref[:, :],
                                  eemb_ref)


def _row2(a):
    return a.reshape(1, -1)


def kernel(embeddings, src_mask, pitch_target, energy_target, pitch_bins,
           energy_bins, pitch_emb, energy_emb, p_params, e_params):
    f32, bf16 = jnp.float32, jnp.bfloat16
    mask_f = src_mask.astype(f32).reshape(B, T, 1)
    pt = pitch_target.reshape(B, T, 1)
    et = energy_target.reshape(B, T, 1)

    inf = jnp.full((1,), jnp.inf, f32)
    plo = jnp.concatenate([-inf, pitch_bins]).reshape(1, NBINS)
    phi = jnp.concatenate([pitch_bins, inf]).reshape(1, NBINS)
    elo = jnp.concatenate([-inf, energy_bins]).reshape(1, NBINS)
    ehi = jnp.concatenate([energy_bins, inf]).reshape(1, NBINS)

    w1 = jnp.concatenate([p_params["conv1_w"].reshape(K * H, FILT),
                          e_params["conv1_w"].reshape(K * H, FILT)],
                         axis=1).astype(bf16)
    b1 = jnp.concatenate([p_params["conv1_b"], e_params["conv1_b"]])

    def head_params(p):
        g1, bt1 = p["ln1_g"], p["ln1_b"]
        w2g = (p["conv2_w"] * g1[None, :, None]).reshape(K * FILT, FILT)
        b2p = p["conv2_b"] + jnp.einsum("c,kcf->f", bt1, p["conv2_w"])
        pad = jnp.where(g1 != 0.0, -bt1 / jnp.where(g1 != 0.0, g1, 1.0), 0.0)
        lw = p["lin_w"][:, 0]
        lwg = lw * p["ln2_g"]
        sc = jnp.stack([jnp.sum(lwg),
                        jnp.sum(p["ln2_b"] * lw) + p["lin_b"][0]])
        return (w2g.astype(bf16), _row2(b2p), _row2(pad), _row2(lwg),
                sc.reshape(1, 2))

    whole = lambda shape: pl.BlockSpec(shape, lambda i: (0,) * len(shape))
    per_b3 = lambda shape: pl.BlockSpec(shape, lambda i: (i, 0, 0))

    in_specs = (
        [per_b3((1, T, H)), per_b3((1, T, 1)), per_b3((1, T, 1)),
         per_b3((1, T, 1))]
        + [whole((1, NBINS))] * 4
        + [whole((NBINS, OUT))] * 2
        + [whole((K * H, 2 * FILT)), whole((1, 2 * FILT))]
        + [whole((K * FILT, FILT)), whole((1, FILT)), whole((1, FILT)),
           whole((1, FILT)), whole((1, 2))] * 2
    )
    out_specs = [per_b3((1, T, 1)), per_b3((1, T, OUT)),
                 per_b3((1, T, 1)), per_b3((1, T, OUT))]
    out_shape = [jax.ShapeDtypeStruct((B, T, 1), f32),
                 jax.ShapeDtypeStruct((B, T, OUT), f32),
                 jax.ShapeDtypeStruct((B, T, 1), f32),
                 jax.ShapeDtypeStruct((B, T, OUT), f32)]

    ppred, pembo, epred, eembo = pl.pallas_call(
        _va_kernel,
        grid=(B,),
        in_specs=in_specs,
        out_specs=out_specs,
        out_shape=out_shape,
    )(embeddings, mask_f, pt, et, plo, phi, elo, ehi,
      pitch_emb.astype(bf16), energy_emb.astype(bf16),
      w1, _row2(b1), *head_params(p_params), *head_params(e_params))

    return (ppred.reshape(B, T), pembo, epred.reshape(B, T), eembo)


# speedup vs baseline: 1.0508x; 1.0068x over previous
import jax
import jax.numpy as jnp
from jax.experimental import pallas as pl
from jax.experimental.pallas import tpu as pltpu

B, T, H = 64, 2048, 256
NBINS, OUT, FILT, K = 256, 256, 256, 3
_EPS = 1e-5


def _im2col3(x, pad_row):
    prv = jnp.concatenate([pad_row, x[:-1]], axis=0)
    nxt = jnp.concatenate([x[1:], pad_row], axis=0)
    return jnp.concatenate([prv, x, nxt], axis=1)


def _rowstats(h):
    mu = jnp.mean(h, axis=-1, keepdims=True)
    m2 = jnp.mean(h * h, axis=-1, keepdims=True)
    return mu, jax.lax.rsqrt(m2 - mu * mu + _EPS)


def _onehot_lookup(v_col, lo_row, hi_row, emb_ref):
    oh = ((v_col > lo_row) & (v_col <= hi_row)).astype(jnp.bfloat16)
    return jnp.dot(oh, emb_ref[:, :], preferred_element_type=jnp.float32)


def _va_kernel(x_ref, mask_ref, pt_ref, et_ref,
               plo_ref, phi_ref, elo_ref, ehi_ref,
               pemb_ref, eemb_ref,
               w1_ref, b1_ref,
               p_w2, p_b2, p_pad, p_lwg, p_sc,
               e_w2, e_b2, e_pad, e_lwg, e_sc,
               ppred_ref, pembo_ref, epred_ref, eembo_ref):
    bf16 = jnp.bfloat16
    x = x_ref[0].astype(bf16)
    mask = mask_ref[0]
    zrow = jnp.zeros((1, H), bf16)

    xs = _im2col3(x, zrow)
    h12 = jnp.dot(xs, w1_ref[:, :], preferred_element_type=jnp.float32)
    h12 = jax.nn.relu(h12 + b1_ref[:, :])

    def head(h, w2, b2, pad, lwg, sc):
        mu, r = _rowstats(h)
        z = (h * r - mu * r).astype(bf16)
        zim = _im2col3(z, pad[:, :].astype(bf16))
        h2 = jnp.dot(zim, w2[:, :], preferred_element_type=jnp.float32)
        h2 = jax.nn.relu(h2 + b2[:, :])
        mu2, r2 = _rowstats(h2)
        s1 = jnp.sum(h2 * lwg[:, :], axis=-1, keepdims=True)
        pred = r2 * (s1 - mu2 * sc[0, 0]) + sc[0, 1]
        return jnp.where(mask > 0.0, 0.0, pred)

    ppred_ref[0] = head(h12[:, :FILT], p_w2, p_b2, p_pad, p_lwg, p_sc)
    epred_ref[0] = head(h12[:, FILT:], e_w2, e_b2, e_pad, e_lwg, e_sc)

    pembo_ref[0] = _onehot_lookup(pt_ref[0], plo_ref[:, :], phi_ref[:, :],
                                  pemb_ref)
    eembo_ref[0] = _onehot_lookup(et_ref[0], elo_ref[:, :], ehi_ref[:, :],
                                  eemb_ref)


def _row2(a):
    return a.reshape(1, -1)


def kernel(embeddings, src_mask, pitch_target, energy_target, pitch_bins,
           energy_bins, pitch_emb, energy_emb, p_params, e_params):
    f32, bf16 = jnp.float32, jnp.bfloat16
    mask_f = src_mask.astype(f32).reshape(B, T, 1)
    pt = pitch_target.reshape(B, T, 1)
    et = energy_target.reshape(B, T, 1)

    inf = jnp.full((1,), jnp.inf, f32)
    plo = jnp.concatenate([-inf, pitch_bins]).reshape(1, NBINS)
    phi = jnp.concatenate([pitch_bins, inf]).reshape(1, NBINS)
    elo = jnp.concatenate([-inf, energy_bins]).reshape(1, NBINS)
    ehi = jnp.concatenate([energy_bins, inf]).reshape(1, NBINS)

    w1 = jnp.concatenate([p_params["conv1_w"].reshape(K * H, FILT),
                          e_params["conv1_w"].reshape(K * H, FILT)],
                         axis=1).astype(bf16)
    b1 = jnp.concatenate([p_params["conv1_b"], e_params["conv1_b"]])

    def head_params(p):
        g1, bt1 = p["ln1_g"], p["ln1_b"]
        w2g = (p["conv2_w"] * g1[None, :, None]).reshape(K * FILT, FILT)
        b2p = p["conv2_b"] + jnp.einsum("c,kcf->f", bt1, p["conv2_w"])
        pad = jnp.where(g1 != 0.0, -bt1 / jnp.where(g1 != 0.0, g1, 1.0), 0.0)
        lw = p["lin_w"][:, 0]
        lwg = lw * p["ln2_g"]
        sc = jnp.stack([jnp.sum(lwg),
                        jnp.sum(p["ln2_b"] * lw) + p["lin_b"][0]])
        return (w2g.astype(bf16), _row2(b2p), _row2(pad), _row2(lwg),
                sc.reshape(1, 2))

    whole = lambda shape: pl.BlockSpec(shape, lambda i: (0,) * len(shape))
    per_b3 = lambda shape: pl.BlockSpec(shape, lambda i: (i, 0, 0))

    in_specs = (
        [per_b3((1, T, H)), per_b3((1, T, 1)), per_b3((1, T, 1)),
         per_b3((1, T, 1))]
        + [whole((1, NBINS))] * 4
        + [whole((NBINS, OUT))] * 2
        + [whole((K * H, 2 * FILT)), whole((1, 2 * FILT))]
        + [whole((K * FILT, FILT)), whole((1, FILT)), whole((1, FILT)),
           whole((1, FILT)), whole((1, 2))] * 2
    )
    out_specs = [per_b3((1, T, 1)), per_b3((1, T, OUT)),
                 per_b3((1, T, 1)), per_b3((1, T, OUT))]
    out_shape = [jax.ShapeDtypeStruct((B, T, 1), f32),
                 jax.ShapeDtypeStruct((B, T, OUT), f32),
                 jax.ShapeDtypeStruct((B, T, 1), f32),
                 jax.ShapeDtypeStruct((B, T, OUT), f32)]

    ppred, pembo, epred, eembo = pl.pallas_call(
        _va_kernel,
        grid=(B,),
        in_specs=in_specs,
        out_specs=out_specs,
        out_shape=out_shape,
        compiler_params=pltpu.CompilerParams(
            vmem_limit_bytes=120 * 1024 * 1024),
    )(embeddings, mask_f, pt, et, plo, phi, elo, ehi,
      pitch_emb.astype(bf16), energy_emb.astype(bf16),
      w1, _row2(b1), *head_params(p_params), *head_params(e_params))

    return (ppred.reshape(B, T), pembo, epred.reshape(B, T), eembo)


# speedup vs baseline: 1.0527x; 1.0018x over previous
import jax
import jax.numpy as jnp
from jax.experimental import pallas as pl
from jax.experimental.pallas import tpu as pltpu

B, T, H = 64, 2048, 256
NBINS, OUT, FILT, K = 256, 256, 256, 3
_EPS = 1e-5


def _im2col3(x, pad_row):
    prv = jnp.concatenate([pad_row, x[:-1]], axis=0)
    nxt = jnp.concatenate([x[1:], pad_row], axis=0)
    return jnp.concatenate([prv, x, nxt], axis=1)


def _rowstats(h):
    mu = jnp.mean(h, axis=-1, keepdims=True)
    m2 = jnp.mean(h * h, axis=-1, keepdims=True)
    return mu, jax.lax.rsqrt(m2 - mu * mu + _EPS)


def _onehot_lookup(v_col, lo_row, hi_row, emb_ref):
    oh = ((v_col > lo_row) & (v_col <= hi_row)).astype(jnp.bfloat16)
    return jnp.dot(oh, emb_ref[:, :], preferred_element_type=jnp.float32)


def _va_kernel(x_ref, mask_ref, pt_ref, et_ref,
               plo_ref, phi_ref, elo_ref, ehi_ref,
               pemb_ref, eemb_ref,
               w1_ref, b1_ref,
               p_w2, p_b2, p_pad, p_lwg, p_sc,
               e_w2, e_b2, e_pad, e_lwg, e_sc,
               ppred_ref, pembo_ref, epred_ref, eembo_ref):
    bf16 = jnp.bfloat16
    x = x_ref[0].astype(bf16)
    mask = mask_ref[0]
    zrow = jnp.zeros((1, H), bf16)

    xs = _im2col3(x, zrow)
    h12 = jnp.dot(xs, w1_ref[:, :], preferred_element_type=jnp.float32)
    h12 = jax.nn.relu(h12 + b1_ref[:, :])

    def head(h, w2, b2, pad, lwg, sc):
        mu, r = _rowstats(h)
        z = (h * r - mu * r).astype(bf16)
        zim = _im2col3(z, pad[:, :].astype(bf16))
        h2 = jnp.dot(zim, w2[:, :], preferred_element_type=jnp.float32)
        h2 = jax.nn.relu(h2 + b2[:, :])
        mu2, r2 = _rowstats(h2)
        s1 = jnp.sum(h2 * lwg[:, :], axis=-1, keepdims=True)
        pred = r2 * (s1 - mu2 * sc[0, 0]) + sc[0, 1]
        return jnp.where(mask > 0.0, 0.0, pred)

    ppred_ref[0] = head(h12[:, :FILT], p_w2, p_b2, p_pad, p_lwg, p_sc)
    epred_ref[0] = head(h12[:, FILT:], e_w2, e_b2, e_pad, e_lwg, e_sc)

    pembo_ref[0] = _onehot_lookup(pt_ref[0], plo_ref[:, :], phi_ref[:, :],
                                  pemb_ref)
    eembo_ref[0] = _onehot_lookup(et_ref[0], elo_ref[:, :], ehi_ref[:, :],
                                  eemb_ref)


def _row2(a):
    return a.reshape(1, -1)


def kernel(embeddings, src_mask, pitch_target, energy_target, pitch_bins,
           energy_bins, pitch_emb, energy_emb, p_params, e_params):
    f32, bf16 = jnp.float32, jnp.bfloat16
    mask_f = src_mask.astype(f32).reshape(B, T, 1)
    pt = pitch_target.reshape(B, T, 1)
    et = energy_target.reshape(B, T, 1)

    inf = jnp.full((1,), jnp.inf, f32)
    plo = jnp.concatenate([-inf, pitch_bins]).reshape(1, NBINS)
    phi = jnp.concatenate([pitch_bins, inf]).reshape(1, NBINS)
    elo = jnp.concatenate([-inf, energy_bins]).reshape(1, NBINS)
    ehi = jnp.concatenate([energy_bins, inf]).reshape(1, NBINS)

    w1 = jnp.concatenate([p_params["conv1_w"].reshape(K * H, FILT),
                          e_params["conv1_w"].reshape(K * H, FILT)],
                         axis=1).astype(bf16)
    b1 = jnp.concatenate([p_params["conv1_b"], e_params["conv1_b"]])

    def head_params(p):
        g1, bt1 = p["ln1_g"], p["ln1_b"]
        w2g = (p["conv2_w"] * g1[None, :, None]).reshape(K * FILT, FILT)
        b2p = p["conv2_b"] + jnp.einsum("c,kcf->f", bt1, p["conv2_w"])
        pad = jnp.where(g1 != 0.0, -bt1 / jnp.where(g1 != 0.0, g1, 1.0), 0.0)
        lw = p["lin_w"][:, 0]
        lwg = lw * p["ln2_g"]
        sc = jnp.stack([jnp.sum(lwg),
                        jnp.sum(p["ln2_b"] * lw) + p["lin_b"][0]])
        return (w2g.astype(bf16), _row2(b2p), _row2(pad), _row2(lwg),
                sc.reshape(1, 2))

    whole = lambda shape: pl.BlockSpec(shape, lambda i: (0,) * len(shape))
    per_b3 = lambda shape: pl.BlockSpec(shape, lambda i: (i, 0, 0))

    in_specs = (
        [per_b3((1, T, H)), per_b3((1, T, 1)), per_b3((1, T, 1)),
         per_b3((1, T, 1))]
        + [whole((1, NBINS))] * 4
        + [whole((NBINS, OUT))] * 2
        + [whole((K * H, 2 * FILT)), whole((1, 2 * FILT))]
        + [whole((K * FILT, FILT)), whole((1, FILT)), whole((1, FILT)),
           whole((1, FILT)), whole((1, 2))] * 2
    )
    out_specs = [per_b3((1, T, 1)), per_b3((1, T, OUT)),
                 per_b3((1, T, 1)), per_b3((1, T, OUT))]
    out_shape = [jax.ShapeDtypeStruct((B, T, 1), f32),
                 jax.ShapeDtypeStruct((B, T, OUT), f32),
                 jax.ShapeDtypeStruct((B, T, 1), f32),
                 jax.ShapeDtypeStruct((B, T, OUT), f32)]

    ppred, pembo, epred, eembo = pl.pallas_call(
        _va_kernel,
        grid=(B,),
        in_specs=in_specs,
        out_specs=out_specs,
        out_shape=out_shape,
        compiler_params=pltpu.CompilerParams(
            dimension_semantics=("parallel",),
            vmem_limit_bytes=120 * 1024 * 1024),
    )(embeddings, mask_f, pt, et, plo, phi, elo, ehi,
      pitch_emb.astype(bf16), energy_emb.astype(bf16),
      w1, _row2(b1), *head_params(p_params), *head_params(e_params))

    return (ppred.reshape(B, T), pembo, epred.reshape(B, T), eembo)
